# baseline (speedup 1.0000x reference)
import functools

import jax
import jax.numpy as jnp
from jax import lax
from jax.experimental import pallas as pl
from jax.experimental.pallas import tpu as pltpu
from jax.experimental.pallas import tpu_sc as plsc

HIDDEN = 64
VOCAB = 1000
NUM_EMB = 32
NC = 2
NS = 16
NW = NC * NS

BATCH = 1024
SEQ = 50
E_PER_W = BATCH // NW


def _fused_mm_body(emb_ref, w_ref, out_ref):
    out_ref[...] = lax.dot_general(
        emb_ref[...], w_ref[...],
        dimension_numbers=(((1,), (1,)), ((), ())),
        preferred_element_type=jnp.float32,
    )


def _fused_table(emb, W):
    return pl.pallas_call(
        _fused_mm_body,
        out_shape=jax.ShapeDtypeStruct((NUM_EMB, VOCAB), jnp.float32),
    )(emb, W)


def _sc_gather_body(
    table_hbm, idx_hbm, out_hbm, table_s, idx_v, rows_a, rows_b, gsem_a, gsem_b
):
    wid = lax.axis_index("s") * NC + lax.axis_index("c")
    ebase = wid * E_PER_W
    sid = lax.axis_index("s")

    @pl.when(sid == 0)
    def _():
        pltpu.sync_copy(table_hbm, table_s)

    plsc.subcore_barrier()
    pltpu.sync_copy(idx_hbm.at[pl.ds(ebase, E_PER_W)], idx_v)

    def gather_src(e):
        return table_s.at[idx_v.at[e]]

    pltpu.async_copy(gather_src(0), rows_a, gsem_a)

    def body(j, carry):
        a = 2 * j
        b = a + 1
        pltpu.make_async_copy(gather_src(a), rows_a, gsem_a).wait()
        pltpu.async_copy(gather_src(b), rows_b, gsem_b)
        pltpu.sync_copy(rows_a, out_hbm.at[ebase + a])
        pltpu.make_async_copy(gather_src(b), rows_b, gsem_b).wait()

        @pl.when(j < E_PER_W // 2 - 1)
        def _():
            pltpu.async_copy(gather_src(a + 2), rows_a, gsem_a)

        pltpu.sync_copy(rows_b, out_hbm.at[ebase + b])
        return carry

    lax.fori_loop(0, E_PER_W // 2, body, 0)


_sc_gather = functools.partial(
    pl.kernel,
    out_type=jax.ShapeDtypeStruct((BATCH, SEQ, VOCAB), jnp.float32),
    mesh=plsc.VectorSubcoreMesh(core_axis_name="c", subcore_axis_name="s"),
    scratch_types=[
        pltpu.VMEM_SHARED((NUM_EMB, VOCAB), jnp.float32),
        pltpu.VMEM((E_PER_W, SEQ), jnp.int32),
        pltpu.VMEM((SEQ, VOCAB), jnp.float32),
        pltpu.VMEM((SEQ, VOCAB), jnp.float32),
        pltpu.SemaphoreType.DMA,
        pltpu.SemaphoreType.DMA,
    ],
    compiler_params=pltpu.CompilerParams(use_tc_tiling_on_sc=False),
)(_sc_gather_body)


def kernel(input_ids, emb_table, W):
    fused = _fused_table(emb_table, W)
    ids = input_ids.astype(jnp.int32)
    return _sc_gather(fused, ids)


# speedup vs baseline: 1.3488x; 1.3488x over previous
import functools

import jax
import jax.numpy as jnp
from jax import lax
from jax.experimental import pallas as pl
from jax.experimental.pallas import tpu as pltpu
from jax.experimental.pallas import tpu_sc as plsc

HIDDEN = 64
VOCAB = 1000
NUM_EMB = 32
EPAD = 128
NC = 2
NS = 16
NW = NC * NS

BATCH = 1024
SEQ = 50
VBLK = 32
LANES = 16


def _fused_mm_body(w_ref, emb_ref, out_ref):
    out_ref[...] = lax.dot_general(
        w_ref[...], emb_ref[...],
        dimension_numbers=(((1,), (1,)), ((), ())),
        preferred_element_type=jnp.float32,
    )


def _fused_table_t(emb_pad, W):
    return pl.pallas_call(
        _fused_mm_body,
        out_shape=jax.ShapeDtypeStruct((VOCAB, EPAD), jnp.float32),
    )(W, emb_pad)


def _sc_tgather_body(
    ft_hbm, ids_hbm, out_hbm,
    ft_v, ids_a, ids_b, buf_a, buf_b, isem_a, isem_b, wsem_a, wsem_b,
):
    wid = lax.axis_index("s") * NC + lax.axis_index("c")
    v0 = jnp.minimum(wid * VBLK, VOCAB - VBLK)

    def ids_desc(l, buf, sem):
        return pltpu.make_async_copy(
            ids_hbm.at[pl.ds(l * BATCH, BATCH)], buf, sem
        )

    def out_desc(l, buf, sem):
        return pltpu.make_async_copy(
            buf, out_hbm.at[l, pl.ds(v0, VBLK)], sem
        )

    ids_desc(0, ids_a, isem_a).start()
    ids_desc(1, ids_b, isem_b).start()
    pltpu.sync_copy(ft_hbm.at[pl.ds(v0 * EPAD, VBLK * EPAD)], ft_v)

    def compute(ids_ref, buf_ref):
        def bloop(i, c):
            off = i * LANES
            ids16 = ids_ref[pl.ds(off, LANES)]
            for vv in range(VBLK):
                idx = ids16 + (vv * EPAD)
                buf_ref[vv, pl.ds(off, LANES)] = plsc.load_gather(ft_v, [idx])
            return c

        lax.fori_loop(0, BATCH // LANES, bloop, 0)

    def body(j, carry):
        l0 = 2 * j
        l1 = l0 + 1

        ids_desc(l0, ids_a, isem_a).wait()

        @pl.when(j > 0)
        def _():
            out_desc(l0 - 2, buf_a, wsem_a).wait()

        compute(ids_a, buf_a)
        out_desc(l0, buf_a, wsem_a).start()

        @pl.when(j < SEQ // 2 - 1)
        def _():
            ids_desc(l0 + 2, ids_a, isem_a).start()

        ids_desc(l1, ids_b, isem_b).wait()

        @pl.when(j > 0)
        def _():
            out_desc(l1 - 2, buf_b, wsem_b).wait()

        compute(ids_b, buf_b)
        out_desc(l1, buf_b, wsem_b).start()

        @pl.when(j < SEQ // 2 - 1)
        def _():
            ids_desc(l1 + 2, ids_b, isem_b).start()

        return carry

    lax.fori_loop(0, SEQ // 2, body, 0)
    out_desc(SEQ - 2, buf_a, wsem_a).wait()
    out_desc(SEQ - 1, buf_b, wsem_b).wait()


_sc_tgather = functools.partial(
    pl.kernel,
    out_type=jax.ShapeDtypeStruct((SEQ, VOCAB, BATCH), jnp.float32),
    mesh=plsc.VectorSubcoreMesh(core_axis_name="c", subcore_axis_name="s"),
    scratch_types=[
        pltpu.VMEM((VBLK * EPAD,), jnp.float32),
        pltpu.VMEM((BATCH,), jnp.int32),
        pltpu.VMEM((BATCH,), jnp.int32),
        pltpu.VMEM((VBLK, BATCH), jnp.float32),
        pltpu.VMEM((VBLK, BATCH), jnp.float32),
        pltpu.SemaphoreType.DMA,
        pltpu.SemaphoreType.DMA,
        pltpu.SemaphoreType.DMA,
        pltpu.SemaphoreType.DMA,
    ],
    compiler_params=pltpu.CompilerParams(
        use_tc_tiling_on_sc=True, needs_layout_passes=False
    ),
)(_sc_tgather_body)


def kernel(input_ids, emb_table, W):
    emb_pad = jnp.pad(emb_table, ((0, EPAD - NUM_EMB), (0, 0)))
    fused_t = _fused_table_t(emb_pad, W)
    ids_t = jnp.swapaxes(input_ids.astype(jnp.int32), 0, 1)
    out_t = _sc_tgather(fused_t.reshape(-1), ids_t.reshape(-1))
    return jnp.transpose(out_t, (2, 0, 1))


# speedup vs baseline: 5.3679x; 3.9797x over previous
import functools

import jax
import jax.numpy as jnp
from jax import lax
from jax.experimental import pallas as pl
from jax.experimental.pallas import tpu as pltpu
from jax.experimental.pallas import tpu_sc as plsc

HIDDEN = 64
VOCAB = 1000
NUM_EMB = 32
EPAD = 128
NC = 2
NS = 16
NW = NC * NS

BATCH = 1024
SEQ = 50
VBLK = 32
LANES = 16


def _fused_mm_body(w_ref, emb_ref, out_ref):
    out_ref[...] = lax.dot_general(
        w_ref[...], emb_ref[...],
        dimension_numbers=(((1,), (1,)), ((), ())),
        preferred_element_type=jnp.float32,
    )


def _fused_table_t(emb_pad, W):
    return pl.pallas_call(
        _fused_mm_body,
        out_shape=jax.ShapeDtypeStruct((VOCAB, EPAD), jnp.float32),
    )(W, emb_pad)


def _sc_tgather_body(
    ft_hbm, ids_hbm, out_hbm,
    ft_v, ids_a, ids_b, buf_a, buf_b, isem_a, isem_b, wsem_a, wsem_b,
):
    wid = lax.axis_index("s") * NC + lax.axis_index("c")
    v0 = jnp.minimum(wid * VBLK, VOCAB - VBLK)

    def ids_desc(l, buf, sem):
        return pltpu.make_async_copy(
            ids_hbm.at[pl.ds(l * BATCH, BATCH)], buf, sem
        )

    def out_desc(l, buf, sem):
        return pltpu.make_async_copy(
            buf, out_hbm.at[l, pl.ds(v0, VBLK)], sem
        )

    ids_desc(0, ids_a, isem_a).start()
    ids_desc(1, ids_b, isem_b).start()
    pltpu.sync_copy(ft_hbm.at[pl.ds(v0 * EPAD, VBLK * EPAD)], ft_v)

    def compute(ids_ref, buf_ref):
        @functools.partial(plsc.parallel_loop, 0, BATCH // LANES)
        def _(i):
            off = i * LANES
            ids16 = ids_ref[pl.ds(off, LANES)]
            vals = [
                plsc.load_gather(ft_v, [ids16 + vv * EPAD])
                for vv in range(VBLK)
            ]
            for vv in range(VBLK):
                buf_ref[vv, pl.ds(off, LANES)] = vals[vv]

    def body(j, carry):
        l0 = 2 * j
        l1 = l0 + 1

        ids_desc(l0, ids_a, isem_a).wait()

        @pl.when(j > 0)
        def _():
            out_desc(l0 - 2, buf_a, wsem_a).wait()

        compute(ids_a, buf_a)
        out_desc(l0, buf_a, wsem_a).start()

        @pl.when(j < SEQ // 2 - 1)
        def _():
            ids_desc(l0 + 2, ids_a, isem_a).start()

        ids_desc(l1, ids_b, isem_b).wait()

        @pl.when(j > 0)
        def _():
            out_desc(l1 - 2, buf_b, wsem_b).wait()

        compute(ids_b, buf_b)
        out_desc(l1, buf_b, wsem_b).start()

        @pl.when(j < SEQ // 2 - 1)
        def _():
            ids_desc(l1 + 2, ids_b, isem_b).start()

        return carry

    lax.fori_loop(0, SEQ // 2, body, 0)
    out_desc(SEQ - 2, buf_a, wsem_a).wait()
    out_desc(SEQ - 1, buf_b, wsem_b).wait()


_sc_tgather = functools.partial(
    pl.kernel,
    out_type=jax.ShapeDtypeStruct((SEQ, VOCAB, BATCH), jnp.float32),
    mesh=plsc.VectorSubcoreMesh(core_axis_name="c", subcore_axis_name="s"),
    scratch_types=[
        pltpu.VMEM((VBLK * EPAD,), jnp.float32),
        pltpu.VMEM((BATCH,), jnp.int32),
        pltpu.VMEM((BATCH,), jnp.int32),
        pltpu.VMEM((VBLK, BATCH), jnp.float32),
        pltpu.VMEM((VBLK, BATCH), jnp.float32),
        pltpu.SemaphoreType.DMA,
        pltpu.SemaphoreType.DMA,
        pltpu.SemaphoreType.DMA,
        pltpu.SemaphoreType.DMA,
    ],
    compiler_params=pltpu.CompilerParams(
        use_tc_tiling_on_sc=True, needs_layout_passes=False
    ),
)(_sc_tgather_body)


def kernel(input_ids, emb_table, W):
    emb_pad = jnp.pad(emb_table, ((0, EPAD - NUM_EMB), (0, 0)))
    fused_t = _fused_table_t(emb_pad, W)
    ids_t = jnp.swapaxes(input_ids.astype(jnp.int32), 0, 1)
    out_t = _sc_tgather(fused_t.reshape(-1), ids_t.reshape(-1))
    return jnp.transpose(out_t, (2, 0, 1))
